# Initial kernel scaffold; baseline (speedup 1.0000x reference)
import jax
import jax.numpy as jnp
from jax.experimental import pallas as pl


def kernel(nodes, senders, receivers, W_left, b_left, W_right):
    raise NotImplementedError("write your pallas kernel here")


# speedup vs baseline: 5.6312x; 5.6312x over previous
import functools

import jax
import jax.numpy as jnp
from jax import lax
from jax.experimental import pallas as pl
from jax.experimental.pallas import tpu as pltpu
from jax.experimental.pallas import tpu_sc as plsc

_NC = 2
_NS = 16
_CH = 80


def _sc_aggregate(nodes, senders, receivers, pad_n):
    n_nodes, d = nodes.shape
    n_edges = senders.shape[0]
    nw = _NC * _NS
    per_w = n_edges // nw
    assert per_w * nw == n_edges and per_w % _CH == 0
    n_ch = per_w // _CH
    rows_per_tile = pad_n // _NS
    assert rows_per_tile % 128 == 0

    mesh = plsc.VectorSubcoreMesh(core_axis_name="c", subcore_axis_name="s")

    @functools.partial(
        pl.kernel,
        mesh=mesh,
        out_type=[
            jax.ShapeDtypeStruct((_NC, pad_n, d), jnp.float32),
            jax.ShapeDtypeStruct((_NC, pad_n), jnp.float32),
        ],
        scratch_types=[
            pltpu.VMEM((_CH,), jnp.int32),
            pltpu.VMEM((_CH,), jnp.int32),
            pltpu.VMEM((_CH, d), jnp.float32),
            pltpu.VMEM((_CH,), jnp.float32),
            pltpu.VMEM((128, d), jnp.float32),
            pltpu.VMEM((rows_per_tile,), jnp.float32),
            pltpu.VMEM_SHARED((pad_n, d), jnp.float32),
            pltpu.VMEM_SHARED((pad_n,), jnp.float32),
            pltpu.SemaphoreType.DMA,
        ],
    )
    def agg(nodes_h, send_h, recv_h, sum_h, cnt_h,
            sidx, ridx, rows, ones, zrows, zcnt, acc, cnt, sem):
        c = lax.axis_index("c")
        s = lax.axis_index("s")
        wid = c * _NS + s

        zero16 = jnp.zeros((16,), jnp.float32)
        one16 = jnp.ones((16,), jnp.float32)

        def zrow_body(i, carry):
            for k in range(d // 16):
                zrows[i, pl.ds(k * 16, 16)] = zero16
            return carry

        lax.fori_loop(0, 128, zrow_body, 0)

        def zcnt_body(i, carry):
            zcnt[pl.ds(i * 16, 16)] = zero16
            return carry

        lax.fori_loop(0, rows_per_tile // 16, zcnt_body, 0)

        for k in range(_CH // 16):
            ones[pl.ds(k * 16, 16)] = one16

        r0 = s * rows_per_tile
        for b in range(rows_per_tile // 128):
            pltpu.sync_copy(zrows, acc.at[pl.ds(r0 + b * 128, 128)])
        pltpu.sync_copy(zcnt, cnt.at[pl.ds(r0, rows_per_tile)])
        plsc.subcore_barrier()

        e0 = wid * per_w

        def chunk_body(j, carry):
            base = e0 + j * _CH
            pltpu.sync_copy(send_h.at[pl.ds(base, _CH)], sidx)
            pltpu.sync_copy(recv_h.at[pl.ds(base, _CH)], ridx)
            pltpu.async_copy(nodes_h.at[sidx], rows, sem).wait()
            pltpu.sync_copy(rows, acc.at[ridx], add=True)
            pltpu.sync_copy(ones, cnt.at[ridx], add=True)
            return carry

        lax.fori_loop(0, n_ch, chunk_body, 0)
        plsc.subcore_barrier()

        pltpu.sync_copy(acc.at[pl.ds(r0, rows_per_tile)],
                        sum_h.at[c, pl.ds(r0, rows_per_tile)])
        pltpu.sync_copy(cnt.at[pl.ds(r0, rows_per_tile)],
                        cnt_h.at[c, pl.ds(r0, rows_per_tile)])

    return agg(nodes, senders, receivers)


def _tc_combine(nodes_p, s0, s1, c0, c1, w_left, b_left, w_right):
    pn, d = nodes_p.shape
    out_ch = w_right.shape[1]
    blk = 1024
    assert pn % blk == 0

    def body(n_ref, s0_ref, s1_ref, c0_ref, c1_ref, wl_ref, b_ref, wr_ref,
             o_ref):
        cnt = jnp.maximum(c0_ref[...] + c1_ref[...], 1.0)
        h_agg = (s0_ref[...] + s1_ref[...]) / cnt
        w_comb = wl_ref[0:d, :] + wr_ref[...]
        out = jnp.dot(n_ref[...], w_comb, preferred_element_type=jnp.float32)
        out = out + jnp.dot(h_agg, wl_ref[d:, :],
                            preferred_element_type=jnp.float32)
        o_ref[...] = out + b_ref[...]

    return pl.pallas_call(
        body,
        grid=(pn // blk,),
        in_specs=[
            pl.BlockSpec((blk, d), lambda i: (i, 0)),
            pl.BlockSpec((blk, d), lambda i: (i, 0)),
            pl.BlockSpec((blk, d), lambda i: (i, 0)),
            pl.BlockSpec((blk, 1), lambda i: (i, 0)),
            pl.BlockSpec((blk, 1), lambda i: (i, 0)),
            pl.BlockSpec((2 * d, out_ch), lambda i: (0, 0)),
            pl.BlockSpec((1, out_ch), lambda i: (0, 0)),
            pl.BlockSpec((d, out_ch), lambda i: (0, 0)),
        ],
        out_specs=pl.BlockSpec((blk, out_ch), lambda i: (i, 0)),
        out_shape=jax.ShapeDtypeStruct((pn, out_ch), jnp.float32),
    )(nodes_p, s0, s1, c0, c1, w_left, b_left, w_right)


def kernel(nodes, senders, receivers, W_left, b_left, W_right):
    n_nodes, d = nodes.shape
    pad_n = ((n_nodes + 1023) // 1024) * 1024
    sums, counts = _sc_aggregate(nodes, senders, receivers, pad_n)
    nodes_p = jnp.pad(nodes, ((0, pad_n - n_nodes), (0, 0)))
    out = _tc_combine(
        nodes_p,
        sums[0], sums[1],
        counts[0].reshape(pad_n, 1), counts[1].reshape(pad_n, 1),
        W_left, b_left.reshape(1, -1), W_right,
    )
    return out[:n_nodes]


# speedup vs baseline: 9.9974x; 1.7754x over previous
import functools

import jax
import jax.numpy as jnp
from jax import lax
from jax.experimental import pallas as pl
from jax.experimental.pallas import tpu as pltpu
from jax.experimental.pallas import tpu_sc as plsc

_NC = 2
_NS = 16
_CH = 80


def _sc_aggregate(nodes, senders3, receivers3, pad_n):
    n_nodes, d = nodes.shape
    nw, n_ch, ch = senders3.shape
    assert nw == _NC * _NS and ch == _CH
    rows_per_tile = pad_n // _NS
    assert rows_per_tile % 128 == 0

    mesh = plsc.VectorSubcoreMesh(core_axis_name="c", subcore_axis_name="s")

    @functools.partial(
        pl.kernel,
        mesh=mesh,
        out_type=[
            jax.ShapeDtypeStruct((_NC, pad_n, d), jnp.float32),
            jax.ShapeDtypeStruct((_NC, pad_n), jnp.float32),
        ],
        scratch_types=[
            pltpu.VMEM((2, _CH), jnp.int32),
            pltpu.VMEM((2, _CH), jnp.int32),
            pltpu.VMEM((_CH, d), jnp.float32),
            pltpu.VMEM((_CH, d), jnp.float32),
            pltpu.VMEM((128,), jnp.float32),
            pltpu.VMEM((32, d), jnp.float32),
            pltpu.VMEM((rows_per_tile,), jnp.float32),
            pltpu.VMEM_SHARED((pad_n, d), jnp.float32),
            pltpu.VMEM_SHARED((pad_n,), jnp.float32),
            pltpu.SemaphoreType.DMA,
            pltpu.SemaphoreType.DMA,
            pltpu.SemaphoreType.DMA,
            pltpu.SemaphoreType.DMA,
        ],
    )
    def agg(nodes_h, send_h, recv_h, sum_h, cnt_h,
            sidx, ridx, rows0, rows1, ones, zrows, zcnt, acc, cnt,
            rsem0, rsem1, isem0, isem1):
        c = lax.axis_index("c")
        s = lax.axis_index("s")
        wid = c * _NS + s
        rows = (rows0, rows1)
        rsems = (rsem0, rsem1)
        isems = (isem0, isem1)

        def idx_start(j, b):
            pltpu.async_copy(send_h.at[wid, j], sidx.at[b], isems[b])
            pltpu.async_copy(recv_h.at[wid, j], ridx.at[b], isems[b])

        def idx_wait(b):
            pltpu.make_async_copy(send_h.at[wid, 0], sidx.at[b],
                                  isems[b]).wait()
            pltpu.make_async_copy(recv_h.at[wid, 0], ridx.at[b],
                                  isems[b]).wait()

        def gather_start(b):
            pltpu.async_copy(nodes_h.at[sidx.at[b]], rows[b], rsems[b])

        def gather_wait(b):
            pltpu.make_async_copy(nodes_h.at[sidx.at[b]], rows[b],
                                  rsems[b]).wait()

        def scatter(b):
            pltpu.sync_copy(rows[b], acc.at[ridx.at[b]], add=True)
            pltpu.sync_copy(ones.at[pl.ds(0, _CH)], cnt.at[ridx.at[b]],
                            add=True)

        idx_start(0, 0)
        idx_start(1, 1)

        zero16 = jnp.zeros((16,), jnp.float32)
        one16 = jnp.ones((16,), jnp.float32)

        def zrow_body(i, carry):
            for k in range(d // 16):
                zrows[i, pl.ds(k * 16, 16)] = zero16
            return carry

        lax.fori_loop(0, 32, zrow_body, 0)

        def zcnt_body(i, carry):
            zcnt[pl.ds(i * 16, 16)] = zero16
            return carry

        lax.fori_loop(0, rows_per_tile // 16, zcnt_body, 0)

        for k in range(128 // 16):
            ones[pl.ds(k * 16, 16)] = one16

        r0 = s * rows_per_tile
        for b in range(rows_per_tile // 32):
            pltpu.sync_copy(zrows, acc.at[pl.ds(r0 + b * 32, 32)])
        pltpu.sync_copy(zcnt, cnt.at[pl.ds(r0, rows_per_tile)])
        plsc.subcore_barrier()

        idx_wait(0)
        gather_start(0)

        def chunk_body(g, carry):
            j0 = 2 * g

            @pl.when(j0 + 1 < n_ch)
            def _():
                idx_wait(1)
                gather_start(1)

            gather_wait(0)
            scatter(0)

            @pl.when(j0 + 2 < n_ch)
            def _():
                idx_start(j0 + 2, 0)
                idx_wait(0)
                gather_start(0)

            @pl.when(j0 + 1 < n_ch)
            def _():
                gather_wait(1)
                scatter(1)

            @pl.when(j0 + 3 < n_ch)
            def _():
                idx_start(j0 + 3, 1)

            return carry

        lax.fori_loop(0, (n_ch + 1) // 2, chunk_body, 0)
        plsc.subcore_barrier()

        pltpu.sync_copy(acc.at[pl.ds(r0, rows_per_tile)],
                        sum_h.at[c, pl.ds(r0, rows_per_tile)])
        pltpu.sync_copy(cnt.at[pl.ds(r0, rows_per_tile)],
                        cnt_h.at[c, pl.ds(r0, rows_per_tile)])

    return agg(nodes, senders3, receivers3)


def _tc_combine(nodes_p, s0, s1, c0, c1, w_left, b_left, w_right):
    pn, d = nodes_p.shape
    out_ch = w_right.shape[1]
    blk = 1024
    assert pn % blk == 0

    def body(n_ref, s0_ref, s1_ref, c0_ref, c1_ref, wl_ref, b_ref, wr_ref,
             o_ref):
        cnt = jnp.maximum(c0_ref[...] + c1_ref[...], 1.0)
        h_agg = (s0_ref[...] + s1_ref[...]) / cnt
        w_comb = wl_ref[0:d, :] + wr_ref[...]
        out = jnp.dot(n_ref[...], w_comb, preferred_element_type=jnp.float32)
        out = out + jnp.dot(h_agg, wl_ref[d:, :],
                            preferred_element_type=jnp.float32)
        o_ref[...] = out + b_ref[...]

    return pl.pallas_call(
        body,
        grid=(pn // blk,),
        in_specs=[
            pl.BlockSpec((blk, d), lambda i: (i, 0)),
            pl.BlockSpec((blk, d), lambda i: (i, 0)),
            pl.BlockSpec((blk, d), lambda i: (i, 0)),
            pl.BlockSpec((blk, 1), lambda i: (i, 0)),
            pl.BlockSpec((blk, 1), lambda i: (i, 0)),
            pl.BlockSpec((2 * d, out_ch), lambda i: (0, 0)),
            pl.BlockSpec((1, out_ch), lambda i: (0, 0)),
            pl.BlockSpec((d, out_ch), lambda i: (0, 0)),
        ],
        out_specs=pl.BlockSpec((blk, out_ch), lambda i: (i, 0)),
        out_shape=jax.ShapeDtypeStruct((pn, out_ch), jnp.float32),
    )(nodes_p, s0, s1, c0, c1, w_left, b_left, w_right)


def kernel(nodes, senders, receivers, W_left, b_left, W_right):
    n_nodes, d = nodes.shape
    n_edges = senders.shape[0]
    nw = _NC * _NS
    per_w = n_edges // nw
    assert per_w * nw == n_edges and per_w % _CH == 0
    pad_n = ((n_nodes + 1023) // 1024) * 1024
    senders3 = senders.reshape(nw, per_w // _CH, _CH)
    receivers3 = receivers.reshape(nw, per_w // _CH, _CH)
    sums, counts = _sc_aggregate(nodes, senders3, receivers3, pad_n)
    nodes_p = jnp.pad(nodes, ((0, pad_n - n_nodes), (0, 0)))
    out = _tc_combine(
        nodes_p,
        sums[0], sums[1],
        counts[0].reshape(pad_n, 1), counts[1].reshape(pad_n, 1),
        W_left, b_left.reshape(1, -1), W_right,
    )
    return out[:n_nodes]
